# baseline (speedup 1.0000x reference)
import functools

import jax
import jax.numpy as jnp
from jax import lax
from jax.experimental import pallas as pl
from jax.experimental.pallas import tpu as pltpu
from jax.experimental.pallas import tpu_sc as plsc

NC = 2
NS = 16
NW = NC * NS
B = 80
DEG_W = 128


def _make_agg(N, N_pad, D, E):
    EPT = E // NW
    NB = EPT // B
    mesh = plsc.VectorSubcoreMesh(core_axis_name="c", subcore_axis_name="s")
    RPT = N_pad // NS

    @functools.partial(
        pl.kernel,
        out_type=jax.ShapeDtypeStruct((NC, N_pad, D), jnp.float32),
        mesh=mesh,
        scratch_types=[
            pltpu.VMEM((EPT,), jnp.int32),
            pltpu.VMEM((NB, B), jnp.int32),
            pltpu.VMEM((B, D), jnp.float32),
            pltpu.VMEM((B, D), jnp.float32),
            pltpu.VMEM_SHARED((N_pad, D), jnp.float32),
            pltpu.SemaphoreType.DMA,
            pltpu.SemaphoreType.DMA,
        ],
    )
    def agg(rows_hbm, src_hbm, dst_hbm, zz_hbm, out_hbm,
            src_v, dst_v, buf0, buf1, acc_sh, sem0, sem1):
        cid = lax.axis_index("c")
        sid = lax.axis_index("s")
        wid = cid * NS + sid
        base = wid * EPT

        pltpu.sync_copy(zz_hbm, acc_sh.at[pl.ds(sid * RPT, RPT)])
        pltpu.sync_copy(src_hbm.at[pl.ds(base, EPT)], src_v)
        pltpu.sync_copy(dst_hbm.at[wid], dst_v)
        plsc.subcore_barrier()

        def gather(g, buf, sem):
            return pltpu.async_copy(
                rows_hbm.at[src_v.at[pl.ds(g * B, B)]], buf, sem)

        def scatter(g, buf):
            pltpu.sync_copy(buf, acc_sh.at[dst_v.at[g]], add=True)

        gather(0, buf0, sem0).wait()
        scatter(0, buf0)

        @pl.loop(1, NB, step=2)
        def _(g):
            c0 = gather(g, buf0, sem0)
            c1 = gather(g + 1, buf1, sem1)
            c0.wait()
            scatter(g, buf0)
            c1.wait()
            scatter(g + 1, buf1)

        plsc.subcore_barrier()

        pltpu.sync_copy(acc_sh.at[pl.ds(sid * RPT, RPT)],
                        out_hbm.at[cid, pl.ds(sid * RPT, RPT)])

    return agg


def _make_deg(N_pad, E):
    EPT = E // NW
    NB = EPT // B
    mesh = plsc.VectorSubcoreMesh(core_axis_name="c", subcore_axis_name="s")
    RPT = N_pad // NS

    @functools.partial(
        pl.kernel,
        out_type=jax.ShapeDtypeStruct((NC, N_pad, DEG_W), jnp.float32),
        mesh=mesh,
        scratch_types=[
            pltpu.VMEM((NB, B), jnp.int32),
            pltpu.VMEM((B, DEG_W), jnp.float32),
            pltpu.VMEM_SHARED((N_pad, DEG_W), jnp.float32),
            pltpu.SemaphoreType.DMA,
            pltpu.SemaphoreType.DMA,
        ],
    )
    def deg(dst_hbm, ones_hbm, zz_hbm, out_hbm, dst_v, ones_v, acc_sh,
            sem0, sem1):
        cid = lax.axis_index("c")
        sid = lax.axis_index("s")
        wid = cid * NS + sid

        pltpu.sync_copy(zz_hbm, acc_sh.at[pl.ds(sid * RPT, RPT)])
        pltpu.sync_copy(ones_hbm, ones_v)
        pltpu.sync_copy(dst_hbm.at[wid], dst_v)
        plsc.subcore_barrier()

        pltpu.async_copy(ones_v, acc_sh.at[dst_v.at[0]], sem0,
                         add=True).wait()

        @pl.loop(1, NB, step=2)
        def _(g):
            c0 = pltpu.async_copy(ones_v, acc_sh.at[dst_v.at[g]], sem0,
                                  add=True)
            c1 = pltpu.async_copy(ones_v, acc_sh.at[dst_v.at[g + 1]], sem1,
                                  add=True)
            c0.wait()
            c1.wait()

        plsc.subcore_barrier()
        pltpu.sync_copy(acc_sh.at[pl.ds(sid * RPT, RPT)],
                        out_hbm.at[cid, pl.ds(sid * RPT, RPT)])

    return deg


def _dis_from(d0, d1):
    deg = d0 + d1
    return jnp.where(deg > 0, lax.rsqrt(jnp.maximum(deg, 1.0)), 0.0)


def _mm(x, W):
    N, D = x.shape

    def body(x_ref, w_ref, h_ref):
        h_ref[...] = jnp.dot(x_ref[...], w_ref[...],
                             preferred_element_type=jnp.float32)

    return pl.pallas_call(
        body,
        out_shape=jax.ShapeDtypeStruct((N, D), jnp.float32),
    )(x, W)


def _scale(h, d0, d1):
    N, D = h.shape

    def body(h_ref, d0_ref, d1_ref, hs_ref, dis_ref):
        dis = _dis_from(d0_ref[...], d1_ref[...])
        hs_ref[...] = h_ref[...] * dis
        dis_ref[...] = dis

    return pl.pallas_call(
        body,
        out_shape=[
            jax.ShapeDtypeStruct((N, D), jnp.float32),
            jax.ShapeDtypeStruct((N, 1), jnp.float32),
        ],
    )(h, d0, d1)


def _layer2(p0, p1, dis, b1, W2):
    N, D = p0.shape

    def body(p0_ref, p1_ref, dis_ref, b1_ref, w_ref, o_ref):
        dis = dis_ref[...]
        t = jax.nn.relu((p0_ref[...] + p1_ref[...]) * dis + b1_ref[...])
        o_ref[...] = jnp.dot(t, w_ref[...],
                             preferred_element_type=jnp.float32) * dis

    return pl.pallas_call(
        body,
        out_shape=jax.ShapeDtypeStruct((N, D), jnp.float32),
    )(p0, p1, dis, b1, W2)


def _final(q0, q1, dis, b2):
    N, D = q0.shape

    def body(q0_ref, q1_ref, dis_ref, b2_ref, o_ref):
        o_ref[...] = (q0_ref[...] + q1_ref[...]) * dis_ref[...] + b2_ref[...]

    return pl.pallas_call(
        body,
        out_shape=jax.ShapeDtypeStruct((N, D), jnp.float32),
    )(q0, q1, dis, b2)


def kernel(x, edge_index, W1, b1, W2, b2):
    N, D = x.shape
    E = edge_index.shape[1]
    N_pad = ((N + 8 * NS - 1) // (8 * NS)) * (8 * NS)

    EPT = E // NW
    src = edge_index[0]
    dstr = edge_index[1].reshape(NW, EPT // B, B)
    ones = jnp.ones((B, DEG_W), jnp.float32)
    zdeg = jnp.zeros((N_pad // NS, DEG_W), jnp.float32)
    zz = jnp.zeros((N_pad // NS, D), jnp.float32)

    deg_p = _make_deg(N_pad, E)(dstr, ones, zdeg)
    d0 = deg_p[0, :N, 0:1]
    d1 = deg_p[1, :N, 0:1]

    h1 = _mm(x, W1)
    hs1, dis = _scale(h1, d0, d1)
    p = _make_agg(N, N_pad, D, E)(hs1, src, dstr, zz)
    hs2 = _layer2(p[0, :N], p[1, :N], dis, b1.reshape(1, D), W2)
    q = _make_agg(N, N_pad, D, E)(hs2, src, dstr, zz)
    out = _final(q[0, :N], q[1, :N], dis, b2.reshape(1, D))
    return out


# speedup vs baseline: 1.0283x; 1.0283x over previous
import functools

import jax
import jax.numpy as jnp
from jax import lax
from jax.experimental import pallas as pl
from jax.experimental.pallas import tpu as pltpu
from jax.experimental.pallas import tpu_sc as plsc

NC = 2
NS = 16
NW = NC * NS
B = 80
DEG_W = 128


def _make_agg(N, N_pad, D, E):
    EPT = E // NW
    NB = EPT // B
    mesh = plsc.VectorSubcoreMesh(core_axis_name="c", subcore_axis_name="s")
    RPT = N_pad // NS

    @functools.partial(
        pl.kernel,
        out_type=jax.ShapeDtypeStruct((NC, N_pad, D), jnp.float32),
        mesh=mesh,
        scratch_types=[
            pltpu.VMEM((EPT,), jnp.int32),
            pltpu.VMEM((NB, B), jnp.int32),
            pltpu.VMEM((B, D), jnp.float32),
            pltpu.VMEM((B, D), jnp.float32),
            pltpu.VMEM_SHARED((N_pad, D), jnp.float32),
            pltpu.SemaphoreType.DMA,
            pltpu.SemaphoreType.DMA,
            pltpu.SemaphoreType.DMA,
            pltpu.SemaphoreType.DMA,
        ],
    )
    def agg(rows_hbm, src_hbm, dst_hbm, zz_hbm, out_hbm,
            src_v, dst_v, buf0, buf1, acc_sh, g0, g1, s0, s1):
        cid = lax.axis_index("c")
        sid = lax.axis_index("s")
        wid = cid * NS + sid
        base = wid * EPT

        pltpu.sync_copy(zz_hbm, acc_sh.at[pl.ds(sid * RPT, RPT)])
        pltpu.sync_copy(src_hbm.at[pl.ds(base, EPT)], src_v)
        pltpu.sync_copy(dst_hbm.at[wid], dst_v)
        plsc.subcore_barrier()

        def gather(g, buf, sem):
            return pltpu.async_copy(
                rows_hbm.at[src_v.at[pl.ds(g * B, B)]], buf, sem)

        def scatter(g, buf, sem):
            pltpu.async_copy(buf, acc_sh.at[dst_v.at[g]], sem, add=True)

        def drain_scatter(g, buf, sem):
            pltpu.make_async_copy(buf, acc_sh.at[dst_v.at[g]], sem).wait()

        c0 = gather(0, buf0, g0)
        c1 = gather(1, buf1, g1)
        c0.wait()
        scatter(0, buf0, s0)
        c1.wait()
        scatter(1, buf1, s1)

        @pl.loop(2, NB - 1, step=2)
        def _(g):
            drain_scatter(g, buf0, s0)
            c0 = gather(g, buf0, g0)
            drain_scatter(g, buf1, s1)
            c1 = gather(g + 1, buf1, g1)
            c0.wait()
            scatter(g, buf0, s0)
            c1.wait()
            scatter(g + 1, buf1, s1)

        drain_scatter(0, buf0, s0)
        cT = gather(NB - 1, buf0, g0)
        cT.wait()
        scatter(NB - 1, buf0, s0)
        drain_scatter(0, buf0, s0)
        drain_scatter(0, buf1, s1)

        plsc.subcore_barrier()

        pltpu.sync_copy(acc_sh.at[pl.ds(sid * RPT, RPT)],
                        out_hbm.at[cid, pl.ds(sid * RPT, RPT)])

    return agg


def _make_deg(N_pad, E):
    EPT = E // NW
    NB = EPT // B
    mesh = plsc.VectorSubcoreMesh(core_axis_name="c", subcore_axis_name="s")
    RPT = N_pad // NS

    @functools.partial(
        pl.kernel,
        out_type=jax.ShapeDtypeStruct((NC, N_pad, DEG_W), jnp.float32),
        mesh=mesh,
        scratch_types=[
            pltpu.VMEM((NB, B), jnp.int32),
            pltpu.VMEM((B, DEG_W), jnp.float32),
            pltpu.VMEM_SHARED((N_pad, DEG_W), jnp.float32),
            pltpu.SemaphoreType.DMA,
            pltpu.SemaphoreType.DMA,
        ],
    )
    def deg(dst_hbm, ones_hbm, zz_hbm, out_hbm, dst_v, ones_v, acc_sh,
            sem0, sem1):
        cid = lax.axis_index("c")
        sid = lax.axis_index("s")
        wid = cid * NS + sid

        pltpu.sync_copy(zz_hbm, acc_sh.at[pl.ds(sid * RPT, RPT)])
        pltpu.sync_copy(ones_hbm, ones_v)
        pltpu.sync_copy(dst_hbm.at[wid], dst_v)
        plsc.subcore_barrier()

        pltpu.async_copy(ones_v, acc_sh.at[dst_v.at[0]], sem0,
                         add=True).wait()

        @pl.loop(1, NB, step=2)
        def _(g):
            c0 = pltpu.async_copy(ones_v, acc_sh.at[dst_v.at[g]], sem0,
                                  add=True)
            c1 = pltpu.async_copy(ones_v, acc_sh.at[dst_v.at[g + 1]], sem1,
                                  add=True)
            c0.wait()
            c1.wait()

        plsc.subcore_barrier()
        pltpu.sync_copy(acc_sh.at[pl.ds(sid * RPT, RPT)],
                        out_hbm.at[cid, pl.ds(sid * RPT, RPT)])

    return deg


def _dis_from(d0, d1):
    deg = d0 + d1
    return jnp.where(deg > 0, lax.rsqrt(jnp.maximum(deg, 1.0)), 0.0)


def _mm(x, W):
    N, D = x.shape

    def body(x_ref, w_ref, h_ref):
        h_ref[...] = jnp.dot(x_ref[...], w_ref[...],
                             preferred_element_type=jnp.float32)

    return pl.pallas_call(
        body,
        out_shape=jax.ShapeDtypeStruct((N, D), jnp.float32),
    )(x, W)


def _scale(h, d0, d1):
    N, D = h.shape

    def body(h_ref, d0_ref, d1_ref, hs_ref, dis_ref):
        dis = _dis_from(d0_ref[...], d1_ref[...])
        hs_ref[...] = h_ref[...] * dis
        dis_ref[...] = dis

    return pl.pallas_call(
        body,
        out_shape=[
            jax.ShapeDtypeStruct((N, D), jnp.float32),
            jax.ShapeDtypeStruct((N, 1), jnp.float32),
        ],
    )(h, d0, d1)


def _layer2(p0, p1, dis, b1, W2):
    N, D = p0.shape

    def body(p0_ref, p1_ref, dis_ref, b1_ref, w_ref, o_ref):
        dis = dis_ref[...]
        t = jax.nn.relu((p0_ref[...] + p1_ref[...]) * dis + b1_ref[...])
        o_ref[...] = jnp.dot(t, w_ref[...],
                             preferred_element_type=jnp.float32) * dis

    return pl.pallas_call(
        body,
        out_shape=jax.ShapeDtypeStruct((N, D), jnp.float32),
    )(p0, p1, dis, b1, W2)


def _final(q0, q1, dis, b2):
    N, D = q0.shape

    def body(q0_ref, q1_ref, dis_ref, b2_ref, o_ref):
        o_ref[...] = (q0_ref[...] + q1_ref[...]) * dis_ref[...] + b2_ref[...]

    return pl.pallas_call(
        body,
        out_shape=jax.ShapeDtypeStruct((N, D), jnp.float32),
    )(q0, q1, dis, b2)


def kernel(x, edge_index, W1, b1, W2, b2):
    N, D = x.shape
    E = edge_index.shape[1]
    N_pad = ((N + 8 * NS - 1) // (8 * NS)) * (8 * NS)

    EPT = E // NW
    src = edge_index[0]
    dstr = edge_index[1].reshape(NW, EPT // B, B)
    ones = jnp.ones((B, DEG_W), jnp.float32)
    zdeg = jnp.zeros((N_pad // NS, DEG_W), jnp.float32)
    zz = jnp.zeros((N_pad // NS, D), jnp.float32)

    deg_p = _make_deg(N_pad, E)(dstr, ones, zdeg)
    d0 = deg_p[0, :N, 0:1]
    d1 = deg_p[1, :N, 0:1]

    h1 = _mm(x, W1)
    hs1, dis = _scale(h1, d0, d1)
    p = _make_agg(N, N_pad, D, E)(hs1, src, dstr, zz)
    hs2 = _layer2(p[0, :N], p[1, :N], dis, b1.reshape(1, D), W2)
    q = _make_agg(N, N_pad, D, E)(hs2, src, dstr, zz)
    out = _final(q[0, :N], q[1, :N], dis, b2.reshape(1, D))
    return out


# speedup vs baseline: 1.1596x; 1.1277x over previous
import dataclasses
import functools

import jax
import jax.numpy as jnp
from jax import lax
from jax.experimental import pallas as pl
from jax.experimental.pallas import tpu as pltpu
from jax.experimental.pallas import tpu_sc as plsc

NC = 2
NS = 16
NW = NC * NS
B = 80
DEG_W = 128


def _make_agg(N, N_pad, D, E):
    EPT = E // NW
    NB = EPT // B
    mesh = plsc.VectorSubcoreMesh(core_axis_name="c", subcore_axis_name="s")
    RPT = N_pad // NS

    @functools.partial(
        pl.kernel,
        out_type=jax.ShapeDtypeStruct((NC, N_pad, D), jnp.float32),
        mesh=mesh,
        scratch_types=[
            pltpu.VMEM((EPT,), jnp.int32),
            pltpu.VMEM((NB, B), jnp.int32),
            pltpu.VMEM((B, D), jnp.float32),
            pltpu.VMEM((B, D), jnp.float32),
            pltpu.VMEM_SHARED((N_pad, D), jnp.float32),
            pltpu.SemaphoreType.DMA,
            pltpu.SemaphoreType.DMA,
        ],
    )
    def agg(rows_hbm, src_hbm, dst_hbm, zz_hbm, out_hbm,
            src_v, dst_v, buf0, buf1, acc_sh, g0, g1):
        cid = lax.axis_index("c")
        sid = lax.axis_index("s")
        wid = cid * NS + sid
        base = wid * EPT

        pltpu.sync_copy(zz_hbm, acc_sh.at[pl.ds(sid * RPT, RPT)])
        pltpu.sync_copy(src_hbm.at[pl.ds(base, EPT)], src_v)
        pltpu.sync_copy(dst_hbm.at[wid], dst_v)
        plsc.subcore_barrier()

        def gather(g, buf, sem):
            return pltpu.async_copy(
                rows_hbm.at[src_v.at[pl.ds(g * B, B)]], buf, sem)

        def scatter(g, buf):
            pltpu.sync_copy(buf, acc_sh.at[dst_v.at[g]], add=True)

        gather(0, buf0, g0).wait()
        scatter(0, buf0)

        @pl.loop(1, NB, step=2)
        def _(g):
            c0 = gather(g, buf0, g0)
            c1 = gather(g + 1, buf1, g1)
            c0.wait()
            scatter(g, buf0)
            c1.wait()
            scatter(g + 1, buf1)

        plsc.subcore_barrier()

        pltpu.sync_copy(acc_sh.at[pl.ds(sid * RPT, RPT)],
                        out_hbm.at[cid, pl.ds(sid * RPT, RPT)])

    return agg


def _make_deg(N, E):
    EPT = E // NW
    mesh = plsc.VectorSubcoreMesh(core_axis_name="c", subcore_axis_name="s")
    cp = pltpu.CompilerParams()
    if "needs_layout_passes" in pltpu.CompilerParams.__dataclass_fields__:
        cp = dataclasses.replace(cp, needs_layout_passes=False)

    @functools.partial(
        pl.kernel,
        out_type=jax.ShapeDtypeStruct((NW, N), jnp.float32),
        mesh=mesh,
        compiler_params=cp,
        scratch_types=[
            pltpu.VMEM((EPT,), jnp.int32),
            pltpu.VMEM((N,), jnp.float32),
        ],
    )
    def deg(dst_hbm, zz_hbm, out_hbm, dst_v, hist_v):
        cid = lax.axis_index("c")
        sid = lax.axis_index("s")
        wid = cid * NS + sid

        pltpu.sync_copy(zz_hbm, hist_v)
        pltpu.sync_copy(dst_hbm.at[pl.ds(wid * EPT, EPT)], dst_v)
        ones16 = jnp.full((16,), 1.0, jnp.float32)

        @pl.loop(0, EPT // 16)
        def _(i):
            idx = dst_v[pl.ds(i * 16, 16)]
            plsc.addupdate_scatter(hist_v, [idx], ones16)

        pltpu.sync_copy(hist_v, out_hbm.at[wid])

    return deg


def _dis_from(d0, d1):
    deg = d0 + d1
    return jnp.where(deg > 0, lax.rsqrt(jnp.maximum(deg, 1.0)), 0.0)


def _mm(x, W):
    N, D = x.shape

    def body(x_ref, w_ref, h_ref):
        h_ref[...] = jnp.dot(x_ref[...], w_ref[...],
                             preferred_element_type=jnp.float32)

    return pl.pallas_call(
        body,
        out_shape=jax.ShapeDtypeStruct((N, D), jnp.float32),
    )(x, W)


def _scale(h, degsT):
    N, D = h.shape

    def body(h_ref, dT_ref, hs_ref, dis_ref):
        deg = jnp.sum(dT_ref[...], axis=1, keepdims=True)
        dis = jnp.where(deg > 0, lax.rsqrt(jnp.maximum(deg, 1.0)), 0.0)
        hs_ref[...] = h_ref[...] * dis
        dis_ref[...] = dis

    return pl.pallas_call(
        body,
        out_shape=[
            jax.ShapeDtypeStruct((N, D), jnp.float32),
            jax.ShapeDtypeStruct((N, 1), jnp.float32),
        ],
    )(h, degsT)


def _layer2(p0, p1, dis, b1, W2):
    N, D = p0.shape

    def body(p0_ref, p1_ref, dis_ref, b1_ref, w_ref, o_ref):
        dis = dis_ref[...]
        t = jax.nn.relu((p0_ref[...] + p1_ref[...]) * dis + b1_ref[...])
        o_ref[...] = jnp.dot(t, w_ref[...],
                             preferred_element_type=jnp.float32) * dis

    return pl.pallas_call(
        body,
        out_shape=jax.ShapeDtypeStruct((N, D), jnp.float32),
    )(p0, p1, dis, b1, W2)


def _final(q0, q1, dis, b2):
    N, D = q0.shape

    def body(q0_ref, q1_ref, dis_ref, b2_ref, o_ref):
        o_ref[...] = (q0_ref[...] + q1_ref[...]) * dis_ref[...] + b2_ref[...]

    return pl.pallas_call(
        body,
        out_shape=jax.ShapeDtypeStruct((N, D), jnp.float32),
    )(q0, q1, dis, b2)


def kernel(x, edge_index, W1, b1, W2, b2):
    N, D = x.shape
    E = edge_index.shape[1]
    N_pad = ((N + 8 * NS - 1) // (8 * NS)) * (8 * NS)

    EPT = E // NW
    src = edge_index[0]
    dst = edge_index[1]
    dstr = edge_index[1].reshape(NW, EPT // B, B)
    zdeg = jnp.zeros((N,), jnp.float32)
    zz = jnp.zeros((N_pad // NS, D), jnp.float32)

    deg_p = _make_deg(N, E)(dst, zdeg)
    degsT = deg_p.T

    h1 = _mm(x, W1)
    hs1, dis = _scale(h1, degsT)
    p = _make_agg(N, N_pad, D, E)(hs1, src, dstr, zz)
    hs2 = _layer2(p[0, :N], p[1, :N], dis, b1.reshape(1, D), W2)
    q = _make_agg(N, N_pad, D, E)(hs2, src, dstr, zz)
    out = _final(q[0, :N], q[1, :N], dis, b2.reshape(1, D))
    return out
